# Initial kernel scaffold; baseline (speedup 1.0000x reference)
import jax
import jax.numpy as jnp
from jax.experimental import pallas as pl


def kernel(instance_feats, memory, pid2idx):
    raise NotImplementedError("write your pallas kernel here")


# speedup vs baseline: 1.3537x; 1.3537x over previous
import functools

import jax
import jax.numpy as jnp
from jax import lax
from jax.experimental import pallas as pl
from jax.experimental.pallas import tpu as pltpu
from jax.experimental.pallas import tpu_sc as plsc

_N = 50000
_C = 10000
_D = 768
_LANES = 16
_NCH = _D // _LANES
_NW = 32
_PER_W = 1568
_B = 32
_NBLK = _PER_W // _B
_STRIDE = _LANES + 1

_EPS_PLAIN_SQ = 0.4 * 0.4
_EPS_MOD_SQ = 0.6 * 0.6


def _sc_body(x_hbm, mem_hbm, pid_hbm, out_hbm,
             idx_v, x_v, r_v, o_v, td_v, tx_v, tm_v, sem):
    w = lax.axis_index("s") * 2 + lax.axis_index("c")
    lane = lax.iota(jnp.int32, _LANES)

    def blk(i, carry):
        base = jnp.minimum(w * _PER_W + i * _B, _N - _B)
        pltpu.sync_copy(pid_hbm.at[pl.ds(base, _B)], idx_v)
        g = pltpu.async_copy(mem_hbm.at[idx_v], r_v, sem)
        f = pltpu.async_copy(x_hbm.at[pl.ds(base, _B)], x_v, sem)
        g.wait()
        f.wait()

        for grp in range(_B // _LANES):
            def row(j16, c2):
                j = grp * _LANES + j16
                accd = jnp.zeros((_LANES,), jnp.float32)
                accx = jnp.zeros((_LANES,), jnp.float32)
                accm = jnp.zeros((_LANES,), jnp.float32)
                for k in range(_NCH):
                    xk = x_v[j, pl.ds(k * _LANES, _LANES)]
                    rk = r_v[j, pl.ds(k * _LANES, _LANES)]
                    accd = accd + xk * rk
                    accx = accx + xk * xk
                    accm = accm + rk * rk
                sidx = _STRIDE * lane + j16
                plsc.store_scatter(td_v, [sidx], accd)
                plsc.store_scatter(tx_v, [sidx], accx)
                plsc.store_scatter(tm_v, [sidx], accm)
                return c2

            lax.fori_loop(0, _LANES, row, 0, unroll=2)

            d = jnp.zeros((_LANES,), jnp.float32)
            xs = jnp.zeros((_LANES,), jnp.float32)
            ms = jnp.zeros((_LANES,), jnp.float32)
            for l in range(_LANES):
                gidx = lane + (_STRIDE * l)
                d = d + plsc.load_gather(td_v, [gidx])
                xs = xs + plsc.load_gather(tx_v, [gidx])
                ms = ms + plsc.load_gather(tm_v, [gidx])

            pos = d > 0.0
            d2 = d * d
            xm = xs * ms
            is_mod = jnp.logical_and(pos, d2 >= _EPS_MOD_SQ * xm)
            is_pla = jnp.logical_and(pos, d2 >= _EPS_PLAIN_SQ * xm)
            cls = 2 - is_pla.astype(jnp.int32) - is_mod.astype(jnp.int32)
            o_v[pl.ds(grp * _LANES, _LANES)] = cls

        pltpu.sync_copy(o_v, out_hbm.at[pl.ds(base, _B)])
        return carry

    lax.fori_loop(0, _NBLK, blk, 0)


@jax.jit
def _run(x, mem, pid):
    mesh = plsc.VectorSubcoreMesh(core_axis_name="c", subcore_axis_name="s")
    k = functools.partial(
        pl.kernel,
        mesh=mesh,
        compiler_params=pltpu.CompilerParams(needs_layout_passes=False),
        out_type=jax.ShapeDtypeStruct((_N,), jnp.int32),
        scratch_types=[
            pltpu.VMEM((_B,), jnp.int32),
            pltpu.VMEM((_B, _D), jnp.float32),
            pltpu.VMEM((_B, _D), jnp.float32),
            pltpu.VMEM((_B,), jnp.int32),
            pltpu.VMEM((_STRIDE * _LANES,), jnp.float32),
            pltpu.VMEM((_STRIDE * _LANES,), jnp.float32),
            pltpu.VMEM((_STRIDE * _LANES,), jnp.float32),
            pltpu.SemaphoreType.DMA,
        ],
    )(_sc_body)
    return k(x, mem, pid)


def kernel(instance_feats, memory, pid2idx):
    return _run(instance_feats, memory, pid2idx.astype(jnp.int32))
